# Initial kernel scaffold; baseline (speedup 1.0000x reference)
import jax
import jax.numpy as jnp
from jax.experimental import pallas as pl


def kernel(x, edge_attr, Ws, bs, Wr, br, We, be, a, edge_index):
    raise NotImplementedError("write your pallas kernel here")


# speedup vs baseline: 3.8376x; 3.8376x over previous
import functools

import jax
import jax.numpy as jnp
from jax import lax
from jax.experimental import pallas as pl
from jax.experimental.pallas import tpu as pltpu
from jax.experimental.pallas import tpu_sc as plsc

N = 10000
E = 320000
D = 128
DE = 4
H = 4
HD = D // H

NC = 2
NS = 16
NW = NC * NS
EPW = E // NW
K = 80
NCHUNK = EPW // K
NP = 10240
ROWS_PER_TILE = NP // NS

_mesh = plsc.VectorSubcoreMesh(core_axis_name="c", subcore_axis_name="s",
                               num_cores=NC, num_subcores=NS)


def _proj_body(x_ref, ws_ref, bs_ref, wr_ref, br_ref, hs_ref, hr_ref):
    xb = x_ref[...]
    hs_ref[...] = jnp.dot(xb, ws_ref[...], preferred_element_type=jnp.float32) + bs_ref[...]
    hr_ref[...] = jnp.dot(xb, wr_ref[...], preferred_element_type=jnp.float32) + br_ref[...]


def _project(x, Ws, bs2, Wr, br2):
    nb = 1000
    grid = (N // nb,)
    return pl.pallas_call(
        _proj_body,
        grid=grid,
        in_specs=[
            pl.BlockSpec((nb, D), lambda i: (i, 0)),
            pl.BlockSpec((D, D), lambda i: (0, 0)),
            pl.BlockSpec((1, D), lambda i: (0, 0)),
            pl.BlockSpec((D, D), lambda i: (0, 0)),
            pl.BlockSpec((1, D), lambda i: (0, 0)),
        ],
        out_specs=[
            pl.BlockSpec((nb, D), lambda i: (i, 0)),
            pl.BlockSpec((nb, D), lambda i: (i, 0)),
        ],
        out_shape=[
            jax.ShapeDtypeStruct((N, D), jnp.float32),
            jax.ShapeDtypeStruct((N, D), jnp.float32),
        ],
    )(x, Ws, bs2, Wr, br2)


@functools.partial(
    pl.kernel,
    out_type=(
        jax.ShapeDtypeStruct((E, D), jnp.float32),
        jax.ShapeDtypeStruct((E, D), jnp.float32),
    ),
    mesh=_mesh,
    scratch_types=[
        pltpu.VMEM((K,), jnp.int32),
        pltpu.VMEM((K,), jnp.int32),
        pltpu.VMEM((K, D), jnp.float32),
        pltpu.VMEM((K, D), jnp.float32),
        pltpu.SemaphoreType.DMA,
        pltpu.SemaphoreType.DMA,
    ],
)
def _gather_edges(hs_hbm, hr_hbm, si_hbm, ri_hbm, sent_hbm, recv_hbm,
                  si_v, ri_v, sbuf, rbuf, sem_s, sem_r):
    wid = lax.axis_index("s") * NC + lax.axis_index("c")
    base = wid * EPW

    @pl.loop(0, NCHUNK)
    def _(ci):
        off = base + ci * K
        pltpu.sync_copy(si_hbm.at[pl.ds(off, K)], si_v)
        pltpu.sync_copy(ri_hbm.at[pl.ds(off, K)], ri_v)
        cs = pltpu.async_copy(hs_hbm.at[si_v], sbuf, sem_s)
        cr = pltpu.async_copy(hr_hbm.at[ri_v], rbuf, sem_r)
        cs.wait()
        cr.wait()
        pltpu.sync_copy(sbuf, sent_hbm.at[pl.ds(off, K)])
        pltpu.sync_copy(rbuf, recv_hbm.at[pl.ds(off, K)])


def _edge_body(sent_ref, recv_ref, ea_ref, we_ref, be_ref, af_ref,
               msg_ref, pp_ref):
    sent = sent_ref[...]
    ea = ea_ref[...]
    he = be_ref[...] + (ea[:, 0:1] * we_ref[0:1, :] + ea[:, 1:2] * we_ref[1:2, :]
                        + ea[:, 2:3] * we_ref[2:3, :] + ea[:, 3:4] * we_ref[3:4, :])
    z = sent + recv_ref[...] + he
    z = jnp.where(z >= 0.0, z, 0.01 * z)
    t = z * af_ref[...]
    ps = []
    for h in range(H):
        sl = slice(HD * h, HD * (h + 1))
        logit = jnp.sum(t[:, sl], axis=1, keepdims=True)
        p = jnp.exp(logit)
        ps.append(p)
        msg_ref[:, sl] = p * sent[:, sl]
    pp_ref[...] = jnp.concatenate(ps + [jnp.zeros((sent.shape[0], D - H), jnp.float32)], axis=1)


def _edge_math(sent, recv, edge_attr, We, be2, a2):
    eb = 2000
    grid = (E // eb,)
    return pl.pallas_call(
        _edge_body,
        grid=grid,
        in_specs=[
            pl.BlockSpec((eb, D), lambda i: (i, 0)),
            pl.BlockSpec((eb, D), lambda i: (i, 0)),
            pl.BlockSpec((eb, DE), lambda i: (i, 0)),
            pl.BlockSpec((DE, D), lambda i: (0, 0)),
            pl.BlockSpec((1, D), lambda i: (0, 0)),
            pl.BlockSpec((1, D), lambda i: (0, 0)),
        ],
        out_specs=[
            pl.BlockSpec((eb, D), lambda i: (i, 0)),
            pl.BlockSpec((eb, D), lambda i: (i, 0)),
        ],
        out_shape=[
            jax.ShapeDtypeStruct((E, D), jnp.float32),
            jax.ShapeDtypeStruct((E, D), jnp.float32),
        ],
    )(sent, recv, edge_attr, We, be2, a2)


RJ = ROWS_PER_TILE // K


@functools.partial(
    pl.kernel,
    out_type=jax.ShapeDtypeStruct((NC * NP, D), jnp.float32),
    mesh=_mesh,
    scratch_types=[
        pltpu.VMEM((K,), jnp.int32),
        pltpu.VMEM((K, D), jnp.float32),
        pltpu.VMEM_SHARED((NP, D), jnp.float32),
        pltpu.SemaphoreType.DMA,
    ],
)
def _scatter_edges(msg_hbm, ri_hbm, zm_hbm, accm_out,
                   ri_v, mbuf, accm_sh, sem_m):
    cid = lax.axis_index("c")
    sid = lax.axis_index("s")
    wid = sid * NC + cid
    base = wid * EPW
    r0 = sid * ROWS_PER_TILE

    pltpu.sync_copy(zm_hbm, mbuf)
    for j in range(RJ):
        pltpu.sync_copy(mbuf, accm_sh.at[pl.ds(r0 + j * K, K)])
    plsc.subcore_barrier()

    @pl.loop(0, NCHUNK)
    def _(ci):
        off = base + ci * K
        pltpu.sync_copy(ri_hbm.at[pl.ds(off, K)], ri_v)
        pltpu.sync_copy(msg_hbm.at[pl.ds(off, K)], mbuf)
        pltpu.sync_copy(mbuf, accm_sh.at[ri_v], add=True)

    plsc.subcore_barrier()
    for j in range(RJ):
        pltpu.sync_copy(accm_sh.at[pl.ds(r0 + j * K, K)], mbuf)
        pltpu.sync_copy(mbuf, accm_out.at[pl.ds(cid * NP + r0 + j * K, K)])


def _final_body(accm_ref, accp_ref, out_ref):
    am = accm_ref[0] + accm_ref[1]
    ap = accp_ref[0] + accp_ref[1]
    for h in range(H):
        sl = slice(HD * h, HD * (h + 1))
        s = ap[:, h:h + 1]
        s = jnp.where(s != 0.0, s, 1.0)
        out_ref[:, sl] = am[:, sl] / s


def _finalize(accm, accp):
    nb = 1000
    grid = (N // nb,)
    return pl.pallas_call(
        _final_body,
        grid=grid,
        in_specs=[
            pl.BlockSpec((NC, nb, D), lambda i: (0, i, 0)),
            pl.BlockSpec((NC, nb, D), lambda i: (0, i, 0)),
        ],
        out_specs=pl.BlockSpec((nb, D), lambda i: (i, 0)),
        out_shape=jax.ShapeDtypeStruct((N, D), jnp.float32),
    )(accm, accp)


def kernel(x, edge_attr, Ws, bs, Wr, br, We, be, a, edge_index):
    senders = edge_index[0]
    receivers = edge_index[1]
    hs, hr = _project(x, Ws, bs.reshape(1, D), Wr, br.reshape(1, D))
    sent, recv = _gather_edges(hs, hr, senders, receivers)
    msg, pp = _edge_math(sent, recv, edge_attr, We, be.reshape(1, D),
                         a.reshape(1, D))
    zm = jnp.zeros((K, D), jnp.float32)
    accm_sc = _scatter_edges(msg, receivers, zm).reshape(NC, NP, D)
    accp_sc = _scatter_edges(pp, receivers, zm).reshape(NC, NP, D)
    return _finalize(accm_sc, accp_sc)
